# Initial kernel scaffold; baseline (speedup 1.0000x reference)
import jax
import jax.numpy as jnp
from jax.experimental import pallas as pl


def kernel(x, Wq, bq, Wk, bk, Wv, bv, Wo, bo):
    raise NotImplementedError("write your pallas kernel here")


# speedup vs baseline: 1.9283x; 1.9283x over previous
import math

import jax
import jax.numpy as jnp
from jax import lax
from jax.experimental import pallas as pl
from jax.experimental.pallas import tpu as pltpu

D = 1024
H = 16
HD = 64
BS = 256
NB = 16
BATCH = 2
SINK_ITERS = 5


def _qkv_kernel(x_ref, w_ref, b_ref, qkv_ref, xmean_ref):
    w = w_ref[...]
    b = b_ref[...]
    outs = []
    for bb in range(BATCH):
        xb = x_ref[:, bb * D:(bb + 1) * D]
        o = lax.dot_general(xb, w, (((1,), (1,)), ((), ())),
                            preferred_element_type=jnp.float32) + b
        outs.append(o)
    qkv_ref[...] = jnp.concatenate(outs, axis=1)
    xmean_ref[...] = jnp.mean(x_ref[...], axis=0, keepdims=True)[None]


def _perm_kernel(xmean_ref, wq_ref, bq_ref, wk_ref, bk_ref, perm_ref):
    inv_sqrt_d = 1.0 / math.sqrt(D)
    cols = []
    for bb in range(BATCH):
        xm = xmean_ref[:, bb * D:(bb + 1) * D]
        qb = lax.dot_general(xm, wq_ref[...], (((1,), (1,)), ((), ())),
                             preferred_element_type=jnp.float32) + bq_ref[...]
        kb = lax.dot_general(xm, wk_ref[...], (((1,), (1,)), ((), ())),
                             preferred_element_type=jnp.float32) + bk_ref[...]
        la = lax.dot_general(qb, kb, (((1,), (1,)), ((), ())),
                             preferred_element_type=jnp.float32) * inv_sqrt_d
        for _ in range(SINK_ITERS):
            m1 = jnp.max(la, axis=1, keepdims=True)
            la = la - (m1 + jnp.log(jnp.sum(jnp.exp(la - m1), axis=1, keepdims=True)))
            m0 = jnp.max(la, axis=0, keepdims=True)
            la = la - (m0 + jnp.log(jnp.sum(jnp.exp(la - m0), axis=0, keepdims=True)))
        p = jnp.exp(la)
        mx = jnp.max(p, axis=1, keepdims=True)
        iota = lax.broadcasted_iota(jnp.int32, (NB, NB), 1)
        idx = jnp.min(jnp.where(p >= mx, iota, NB), axis=1, keepdims=True)
        cols.append(idx)
    perm_ref[...] = jnp.concatenate(cols, axis=1)


def _attn_kernel(perm_ref, qkv_ref, wo_ref, bo_ref, out_ref):
    del perm_ref
    scale = HD ** -0.5
    outs = []
    for h in range(H):
        q = qkv_ref[:, h * HD:(h + 1) * HD]
        k = qkv_ref[:, D + h * HD:D + (h + 1) * HD]
        v = qkv_ref[:, 2 * D + h * HD:2 * D + (h + 1) * HD]
        s = lax.dot_general(q, k, (((1,), (1,)), ((), ())),
                            preferred_element_type=jnp.float32) * scale
        m = jnp.max(s, axis=1, keepdims=True)
        e = jnp.exp(s - m)
        p = e / jnp.sum(e, axis=1, keepdims=True)
        outs.append(lax.dot_general(p, v, (((1,), (0,)), ((), ())),
                                    preferred_element_type=jnp.float32))
    cat = jnp.concatenate(outs, axis=1)
    out_ref[...] = lax.dot_general(cat, wo_ref[...], (((1,), (1,)), ((), ())),
                                   preferred_element_type=jnp.float32) + bo_ref[...]


def kernel(x, Wq, bq, Wk, bk, Wv, bv, Wo, bo):
    S, B, Dd = x.shape
    assert (B, Dd) == (BATCH, D) and S == NB * BS

    x2 = x.reshape(S, B * D)
    Wqkv = jnp.concatenate([Wq, Wk, Wv], axis=0)
    bqkv = jnp.concatenate([bq, bk, bv]).reshape(1, 3 * D)

    qkv, xmean3 = pl.pallas_call(
        _qkv_kernel,
        grid=(NB,),
        in_specs=[
            pl.BlockSpec((BS, B * D), lambda i: (i, 0)),
            pl.BlockSpec((3 * D, D), lambda i: (0, 0)),
            pl.BlockSpec((1, 3 * D), lambda i: (0, 0)),
        ],
        out_specs=[
            pl.BlockSpec((BS, B * 3 * D), lambda i: (i, 0)),
            pl.BlockSpec((1, 1, B * D), lambda i: (i, 0, 0)),
        ],
        out_shape=[
            jax.ShapeDtypeStruct((S, B * 3 * D), jnp.float32),
            jax.ShapeDtypeStruct((NB, 1, B * D), jnp.float32),
        ],
    )(x2, Wqkv, bqkv)

    xmean = xmean3.reshape(NB, B * D)
    perm2 = pl.pallas_call(
        _perm_kernel,
        in_specs=[
            pl.BlockSpec((NB, B * D), lambda: (0, 0)),
            pl.BlockSpec((D, D), lambda: (0, 0)),
            pl.BlockSpec((1, D), lambda: (0, 0)),
            pl.BlockSpec((D, D), lambda: (0, 0)),
            pl.BlockSpec((1, D), lambda: (0, 0)),
        ],
        out_specs=pl.BlockSpec((NB, B), lambda: (0, 0)),
        out_shape=jax.ShapeDtypeStruct((NB, B), jnp.int32),
    )(xmean, Wq, bq.reshape(1, D), Wk, bk.reshape(1, D))

    perm = perm2.T.reshape(B * NB)

    grid_spec = pltpu.PrefetchScalarGridSpec(
        num_scalar_prefetch=1,
        grid=(B * NB,),
        in_specs=[
            pl.BlockSpec((BS, 3 * D), lambda t, p: (p[t], t // NB)),
            pl.BlockSpec((D, D), lambda t, p: (0, 0)),
            pl.BlockSpec((1, D), lambda t, p: (0, 0)),
        ],
        out_specs=pl.BlockSpec((BS, D), lambda t, p: (t % NB, t // NB)),
    )
    out_flat = pl.pallas_call(
        _attn_kernel,
        grid_spec=grid_spec,
        out_shape=jax.ShapeDtypeStruct((S, B * D), jnp.float32),
    )(perm, qkv, Wo, bo.reshape(1, D))

    return out_flat.reshape(S, B, D)


# speedup vs baseline: 1.9864x; 1.0301x over previous
import math

import jax
import jax.numpy as jnp
from jax import lax
from jax.experimental import pallas as pl
from jax.experimental.pallas import tpu as pltpu

D = 1024
H = 16
HD = 64
BS = 256
NB = 16
BATCH = 2
SINK_ITERS = 5


def _qkv_kernel(x_ref, w_ref, b_ref, qkv_ref, xmean_ref):
    w = w_ref[...]
    b = b_ref[...]
    outs = []
    for bb in range(BATCH):
        xb = x_ref[:, bb * D:(bb + 1) * D].astype(jnp.bfloat16)
        o = lax.dot_general(xb, w, (((1,), (1,)), ((), ())),
                            preferred_element_type=jnp.float32) + b
        outs.append(o.astype(jnp.bfloat16))
    qkv_ref[...] = jnp.concatenate(outs, axis=1)
    xmean_ref[...] = jnp.mean(x_ref[...], axis=0, keepdims=True)[None]


def _perm_kernel(xmean_ref, wq_ref, bq_ref, wk_ref, bk_ref, perm_ref):
    inv_sqrt_d = 1.0 / math.sqrt(D)
    cols = []
    for bb in range(BATCH):
        xm = xmean_ref[:, bb * D:(bb + 1) * D]
        qb = lax.dot_general(xm, wq_ref[...], (((1,), (1,)), ((), ())),
                             preferred_element_type=jnp.float32) + bq_ref[...]
        kb = lax.dot_general(xm, wk_ref[...], (((1,), (1,)), ((), ())),
                             preferred_element_type=jnp.float32) + bk_ref[...]
        la = lax.dot_general(qb, kb, (((1,), (1,)), ((), ())),
                             preferred_element_type=jnp.float32) * inv_sqrt_d
        for _ in range(SINK_ITERS):
            m1 = jnp.max(la, axis=1, keepdims=True)
            la = la - (m1 + jnp.log(jnp.sum(jnp.exp(la - m1), axis=1, keepdims=True)))
            m0 = jnp.max(la, axis=0, keepdims=True)
            la = la - (m0 + jnp.log(jnp.sum(jnp.exp(la - m0), axis=0, keepdims=True)))
        p = jnp.exp(la)
        mx = jnp.max(p, axis=1, keepdims=True)
        iota = lax.broadcasted_iota(jnp.int32, (NB, NB), 1)
        idx = jnp.min(jnp.where(p >= mx, iota, NB), axis=1, keepdims=True)
        cols.append(idx)
    perm_ref[...] = jnp.concatenate(cols, axis=1)


def _attn_kernel(perm_ref, qkv_ref, wo_ref, bo_ref, out_ref):
    del perm_ref
    scale = HD ** -0.5
    outs = []
    for h in range(H):
        q = qkv_ref[:, h * HD:(h + 1) * HD]
        k = qkv_ref[:, D + h * HD:D + (h + 1) * HD]
        v = qkv_ref[:, 2 * D + h * HD:2 * D + (h + 1) * HD]
        s = lax.dot_general(q, k, (((1,), (1,)), ((), ())),
                            preferred_element_type=jnp.float32) * scale
        m = jnp.max(s, axis=1, keepdims=True)
        e = jnp.exp(s - m)
        p = (e / jnp.sum(e, axis=1, keepdims=True)).astype(jnp.bfloat16)
        outs.append(lax.dot_general(p, v, (((1,), (0,)), ((), ())),
                                    preferred_element_type=jnp.float32).astype(jnp.bfloat16))
    cat = jnp.concatenate(outs, axis=1)
    out_ref[...] = lax.dot_general(cat, wo_ref[...], (((1,), (1,)), ((), ())),
                                   preferred_element_type=jnp.float32) + bo_ref[...]


def kernel(x, Wq, bq, Wk, bk, Wv, bv, Wo, bo):
    S, B, Dd = x.shape
    assert (B, Dd) == (BATCH, D) and S == NB * BS

    x2 = x.reshape(S, B * D)
    Wqkv = jnp.concatenate([Wq, Wk, Wv], axis=0).astype(jnp.bfloat16)
    bqkv = jnp.concatenate([bq, bk, bv]).reshape(1, 3 * D)
    Wo_bf = Wo.astype(jnp.bfloat16)

    qkv, xmean3 = pl.pallas_call(
        _qkv_kernel,
        grid=(NB,),
        in_specs=[
            pl.BlockSpec((BS, B * D), lambda i: (i, 0)),
            pl.BlockSpec((3 * D, D), lambda i: (0, 0)),
            pl.BlockSpec((1, 3 * D), lambda i: (0, 0)),
        ],
        out_specs=[
            pl.BlockSpec((BS, B * 3 * D), lambda i: (i, 0)),
            pl.BlockSpec((1, 1, B * D), lambda i: (i, 0, 0)),
        ],
        out_shape=[
            jax.ShapeDtypeStruct((S, B * 3 * D), jnp.bfloat16),
            jax.ShapeDtypeStruct((NB, 1, B * D), jnp.float32),
        ],
    )(x2, Wqkv, bqkv)

    xmean = xmean3.reshape(NB, B * D)
    perm2 = pl.pallas_call(
        _perm_kernel,
        in_specs=[
            pl.BlockSpec((NB, B * D), lambda: (0, 0)),
            pl.BlockSpec((D, D), lambda: (0, 0)),
            pl.BlockSpec((1, D), lambda: (0, 0)),
            pl.BlockSpec((D, D), lambda: (0, 0)),
            pl.BlockSpec((1, D), lambda: (0, 0)),
        ],
        out_specs=pl.BlockSpec((NB, B), lambda: (0, 0)),
        out_shape=jax.ShapeDtypeStruct((NB, B), jnp.int32),
    )(xmean, Wq, bq.reshape(1, D), Wk, bk.reshape(1, D))

    perm = perm2.T.reshape(B * NB)

    grid_spec = pltpu.PrefetchScalarGridSpec(
        num_scalar_prefetch=1,
        grid=(B * NB,),
        in_specs=[
            pl.BlockSpec((BS, 3 * D), lambda t, p: (p[t], t // NB)),
            pl.BlockSpec((D, D), lambda t, p: (0, 0)),
            pl.BlockSpec((1, D), lambda t, p: (0, 0)),
        ],
        out_specs=pl.BlockSpec((BS, D), lambda t, p: (t % NB, t // NB)),
    )
    out_flat = pl.pallas_call(
        _attn_kernel,
        grid_spec=grid_spec,
        out_shape=jax.ShapeDtypeStruct((S, B * D), jnp.float32),
    )(perm, qkv, Wo_bf, bo.reshape(1, D))

    return out_flat.reshape(S, B, D)


# speedup vs baseline: 2.2749x; 1.1452x over previous
import math

import jax
import jax.numpy as jnp
from jax import lax
from jax.experimental import pallas as pl
from jax.experimental.pallas import tpu as pltpu

D = 1024
H = 16
HD = 64
BS = 256
NB = 16
BATCH = 2
SINK_ITERS = 5


def _perm_kernel(x_ref, wq_ref, bq_ref, wk_ref, bk_ref, perm_ref, xsum_ref):
    i = pl.program_id(0)
    xsum_ref[pl.ds(i, 1), :] = jnp.mean(x_ref[...], axis=0, keepdims=True)

    @pl.when(i == NB - 1)
    def _():
        inv_sqrt_d = 1.0 / math.sqrt(D)
        cols = []
        for bb in range(BATCH):
            xm = xsum_ref[:, bb * D:(bb + 1) * D]
            qb = lax.dot_general(xm, wq_ref[...], (((1,), (1,)), ((), ())),
                                 preferred_element_type=jnp.float32) + bq_ref[...]
            kb = lax.dot_general(xm, wk_ref[...], (((1,), (1,)), ((), ())),
                                 preferred_element_type=jnp.float32) + bk_ref[...]
            la = lax.dot_general(qb, kb, (((1,), (1,)), ((), ())),
                                 preferred_element_type=jnp.float32) * inv_sqrt_d
            for _ in range(SINK_ITERS):
                m1 = jnp.max(la, axis=1, keepdims=True)
                la = la - (m1 + jnp.log(jnp.sum(jnp.exp(la - m1), axis=1, keepdims=True)))
                m0 = jnp.max(la, axis=0, keepdims=True)
                la = la - (m0 + jnp.log(jnp.sum(jnp.exp(la - m0), axis=0, keepdims=True)))
            p = jnp.exp(la)
            mx = jnp.max(p, axis=1, keepdims=True)
            iota = lax.broadcasted_iota(jnp.int32, (NB, NB), 1)
            idx = jnp.min(jnp.where(p >= mx, iota, NB), axis=1, keepdims=True)
            cols.append(idx)
        perm_ref[...] = jnp.concatenate(cols, axis=1)


def _fused_kernel(p_ref, x_ref, wq_ref, wk_ref, wv_ref, wo_ref,
                  bq_ref, bk_ref, bv_ref, bo_ref, out_ref, wbf_ref):
    del p_ref
    t = pl.program_id(0)

    @pl.when(t == 0)
    def _():
        wbf_ref[0 * D:1 * D, :] = wq_ref[...].astype(jnp.bfloat16)
        wbf_ref[1 * D:2 * D, :] = wk_ref[...].astype(jnp.bfloat16)
        wbf_ref[2 * D:3 * D, :] = wv_ref[...].astype(jnp.bfloat16)
        wbf_ref[3 * D:4 * D, :] = wo_ref[...].astype(jnp.bfloat16)

    xb = x_ref[...].astype(jnp.bfloat16)

    def proj(w_idx, b_ref):
        w = wbf_ref[w_idx * D:(w_idx + 1) * D, :]
        o = lax.dot_general(xb, w, (((1,), (1,)), ((), ())),
                            preferred_element_type=jnp.float32) + b_ref[...]
        return o.astype(jnp.bfloat16)

    q = proj(0, bq_ref)
    k = proj(1, bk_ref)
    v = proj(2, bv_ref)

    scale = HD ** -0.5
    outs = []
    for h in range(H):
        qh = q[:, h * HD:(h + 1) * HD]
        kh = k[:, h * HD:(h + 1) * HD]
        vh = v[:, h * HD:(h + 1) * HD]
        s = lax.dot_general(qh, kh, (((1,), (1,)), ((), ())),
                            preferred_element_type=jnp.float32) * scale
        m = jnp.max(s, axis=1, keepdims=True)
        e = jnp.exp(s - m)
        patt = (e / jnp.sum(e, axis=1, keepdims=True)).astype(jnp.bfloat16)
        outs.append(lax.dot_general(patt, vh, (((1,), (0,)), ((), ())),
                                    preferred_element_type=jnp.float32).astype(jnp.bfloat16))
    cat = jnp.concatenate(outs, axis=1)
    wo = wbf_ref[3 * D:4 * D, :]
    out_ref[...] = lax.dot_general(cat, wo, (((1,), (1,)), ((), ())),
                                   preferred_element_type=jnp.float32) + bo_ref[...]


def kernel(x, Wq, bq, Wk, bk, Wv, bv, Wo, bo):
    S, B, Dd = x.shape
    assert (B, Dd) == (BATCH, D) and S == NB * BS

    x2 = x.reshape(S, B * D)
    bq2 = bq.reshape(1, D)
    bk2 = bk.reshape(1, D)
    bv2 = bv.reshape(1, D)
    bo2 = bo.reshape(1, D)

    perm2 = pl.pallas_call(
        _perm_kernel,
        grid=(NB,),
        in_specs=[
            pl.BlockSpec((BS, B * D), lambda i: (i, 0)),
            pl.BlockSpec((D, D), lambda i: (0, 0)),
            pl.BlockSpec((1, D), lambda i: (0, 0)),
            pl.BlockSpec((D, D), lambda i: (0, 0)),
            pl.BlockSpec((1, D), lambda i: (0, 0)),
        ],
        out_specs=pl.BlockSpec((NB, B), lambda i: (0, 0)),
        out_shape=jax.ShapeDtypeStruct((NB, B), jnp.int32),
        scratch_shapes=[pltpu.VMEM((NB, B * D), jnp.float32)],
    )(x2, Wq, bq2, Wk, bk2)

    grid_spec = pltpu.PrefetchScalarGridSpec(
        num_scalar_prefetch=1,
        grid=(B * NB,),
        in_specs=[
            pl.BlockSpec((BS, D), lambda t, p: (p[t % NB, t // NB], t // NB)),
            pl.BlockSpec((D, D), lambda t, p: (0, 0)),
            pl.BlockSpec((D, D), lambda t, p: (0, 0)),
            pl.BlockSpec((D, D), lambda t, p: (0, 0)),
            pl.BlockSpec((D, D), lambda t, p: (0, 0)),
            pl.BlockSpec((1, D), lambda t, p: (0, 0)),
            pl.BlockSpec((1, D), lambda t, p: (0, 0)),
            pl.BlockSpec((1, D), lambda t, p: (0, 0)),
            pl.BlockSpec((1, D), lambda t, p: (0, 0)),
        ],
        out_specs=pl.BlockSpec((BS, D), lambda t, p: (t % NB, t // NB)),
        scratch_shapes=[pltpu.VMEM((4 * D, D), jnp.bfloat16)],
    )
    out_flat = pl.pallas_call(
        _fused_kernel,
        grid_spec=grid_spec,
        out_shape=jax.ShapeDtypeStruct((S, B * D), jnp.float32),
    )(perm2, x2, Wq, Wk, Wv, Wo, bq2, bk2, bv2, bo2)

    return out_flat.reshape(S, B, D)
